# Initial kernel scaffold; baseline (speedup 1.0000x reference)
import jax
import jax.numpy as jnp
from jax.experimental import pallas as pl


def kernel(x, gating_output, gate_up_proj, down_proj):
    raise NotImplementedError("write your pallas kernel here")


# speedup vs baseline: 1.5696x; 1.5696x over previous
import functools

import jax
import jax.numpy as jnp
from jax.experimental import pallas as pl
from jax.experimental.pallas import tpu as pltpu

T = 64
D = 1024
E = 64
DFF = 1024


def _moe_step(x_ref, gate_ref, gup_ref, dp_ref, out_ref, w_ref):
    i = pl.program_id(0)

    @pl.when(i == 0)
    def _routing():
        g = gate_ref[...]
        idx = jax.lax.broadcasted_iota(jnp.int32, (T, E), 1)
        m1 = jnp.max(g, axis=-1, keepdims=True)
        a1 = jnp.min(jnp.where(g == m1, idx, E), axis=-1, keepdims=True)
        g2 = jnp.where(idx == a1, -jnp.inf, g)
        m2 = jnp.max(g2, axis=-1, keepdims=True)
        a2 = jnp.min(jnp.where(g2 == m2, idx, E), axis=-1, keepdims=True)
        p2 = jnp.exp(m2 - m1)
        w1 = 1.0 / (1.0 + p2)
        w2 = p2 / (1.0 + p2)
        w_ref[...] = jnp.where(idx == a1, w1, 0.0) + jnp.where(idx == a2, w2, 0.0)
        out_ref[...] = jnp.zeros_like(out_ref)

    x = x_ref[...]
    w1e = gup_ref[0]
    gu = jax.lax.dot_general(
        x, w1e, (((1,), (1,)), ((), ())),
        preferred_element_type=jnp.float32,
        precision=jax.lax.Precision.HIGHEST)
    gate = gu[:, :DFF]
    up = gu[:, DFF:]
    h = gate * jax.nn.sigmoid(gate) * up
    oe = jax.lax.dot_general(
        h, dp_ref[0], (((1,), (1,)), ((), ())),
        preferred_element_type=jnp.float32,
        precision=jax.lax.Precision.HIGHEST)
    eidx = jax.lax.broadcasted_iota(jnp.int32, (T, E), 1)
    we = jnp.sum(jnp.where(eidx == i, w_ref[...], 0.0), axis=1, keepdims=True)
    out_ref[...] += we * oe


@jax.jit
def kernel(x, gating_output, gate_up_proj, down_proj):
    return pl.pallas_call(
        _moe_step,
        grid=(E,),
        in_specs=[
            pl.BlockSpec((T, D), lambda i: (0, 0)),
            pl.BlockSpec((T, E), lambda i: (0, 0)),
            pl.BlockSpec((1, 2 * DFF, D), lambda i: (i, 0, 0)),
            pl.BlockSpec((1, D, DFF), lambda i: (i, 0, 0)),
        ],
        out_specs=pl.BlockSpec((T, D), lambda i: (0, 0)),
        out_shape=jax.ShapeDtypeStruct((T, D), jnp.float32),
        scratch_shapes=[pltpu.VMEM((T, E), jnp.float32)],
    )(x, gating_output, gate_up_proj, down_proj)


# speedup vs baseline: 4.7252x; 3.0105x over previous
import functools

import jax
import jax.numpy as jnp
from jax.experimental import pallas as pl
from jax.experimental.pallas import tpu as pltpu

T = 64
D = 1024
E = 64
DFF = 1024


def _moe_step(x_ref, gate_ref, gup_ref, dp_ref, out_ref, w_ref):
    i = pl.program_id(0)

    @pl.when(i == 0)
    def _routing():
        g = gate_ref[...]
        idx = jax.lax.broadcasted_iota(jnp.int32, (T, E), 1)
        m1 = jnp.max(g, axis=-1, keepdims=True)
        a1 = jnp.min(jnp.where(g == m1, idx, E), axis=-1, keepdims=True)
        g2 = jnp.where(idx == a1, -jnp.inf, g)
        m2 = jnp.max(g2, axis=-1, keepdims=True)
        a2 = jnp.min(jnp.where(g2 == m2, idx, E), axis=-1, keepdims=True)
        p2 = jnp.exp(m2 - m1)
        w1 = 1.0 / (1.0 + p2)
        w2 = p2 / (1.0 + p2)
        w_ref[...] = jnp.where(idx == a1, w1, 0.0) + jnp.where(idx == a2, w2, 0.0)
        out_ref[...] = jnp.zeros_like(out_ref)

    x = x_ref[...]
    w1e = gup_ref[0]
    gu = jax.lax.dot_general(
        x, w1e, (((1,), (1,)), ((), ())),
        preferred_element_type=jnp.float32,
        precision=jax.lax.Precision.DEFAULT)
    gate = gu[:, :DFF]
    up = gu[:, DFF:]
    h = gate * jax.nn.sigmoid(gate) * up
    oe = jax.lax.dot_general(
        h, dp_ref[0], (((1,), (1,)), ((), ())),
        preferred_element_type=jnp.float32,
        precision=jax.lax.Precision.DEFAULT)
    eidx = jax.lax.broadcasted_iota(jnp.int32, (T, E), 1)
    we = jnp.sum(jnp.where(eidx == i, w_ref[...], 0.0), axis=1, keepdims=True)
    out_ref[...] += we * oe


@jax.jit
def kernel(x, gating_output, gate_up_proj, down_proj):
    return pl.pallas_call(
        _moe_step,
        grid=(E,),
        in_specs=[
            pl.BlockSpec((T, D), lambda i: (0, 0)),
            pl.BlockSpec((T, E), lambda i: (0, 0)),
            pl.BlockSpec((1, 2 * DFF, D), lambda i: (i, 0, 0)),
            pl.BlockSpec((1, D, DFF), lambda i: (i, 0, 0)),
        ],
        out_specs=pl.BlockSpec((T, D), lambda i: (0, 0)),
        out_shape=jax.ShapeDtypeStruct((T, D), jnp.float32),
        scratch_shapes=[pltpu.VMEM((T, E), jnp.float32)],
    )(x, gating_output, gate_up_proj, down_proj)


# speedup vs baseline: 5.2994x; 1.1215x over previous
import jax
import jax.numpy as jnp
from jax.experimental import pallas as pl
from jax.experimental.pallas import tpu as pltpu

T = 64
D = 1024
E = 64
DFF = 1024


def _routing_step(gate_ref, w_ref, sched_ref):
    g = gate_ref[...]
    idx = jax.lax.broadcasted_iota(jnp.int32, (T, E), 1)
    m1 = jnp.max(g, axis=-1, keepdims=True)
    a1 = jnp.min(jnp.where(g == m1, idx, E), axis=-1, keepdims=True)
    g2 = jnp.where(idx == a1, -jnp.inf, g)
    m2 = jnp.max(g2, axis=-1, keepdims=True)
    a2 = jnp.min(jnp.where(g2 == m2, idx, E), axis=-1, keepdims=True)
    p2 = jnp.exp(m2 - m1)
    w1 = 1.0 / (1.0 + p2)
    w2 = p2 / (1.0 + p2)
    w = jnp.where(idx == a1, w1, 0.0) + jnp.where(idx == a2, w2, 0.0)
    w_ref[...] = w

    active = (jnp.sum(jnp.where(w > 0.0, 1, 0), axis=0, keepdims=True) > 0)
    active_i = active.astype(jnp.int32)
    je = jax.lax.broadcasted_iota(jnp.int32, (E, E), 0)
    ee = jax.lax.broadcasted_iota(jnp.int32, (E, E), 1)
    pos = jnp.sum(jnp.where(je <= ee, active_i[0][:, None], 0), axis=0)[None, :] - 1
    num_active = jnp.sum(active_i)
    lane = jax.lax.broadcasted_iota(jnp.int32, (1, E), 1)
    onehot = (pos[0][None, :] == lane[0][:, None]) & active[0][None, :]
    order = jnp.sum(jnp.where(onehot, lane[0][None, :], 0), axis=1)[None, :]
    last = jnp.sum(jnp.where((pos == num_active - 1) & active, lane, 0))
    valid = (lane < num_active)
    order = jnp.where(valid, order, last)
    sched_ref[...] = jnp.concatenate(
        [order, valid.astype(jnp.int32), jnp.zeros((6, E), jnp.int32)], axis=0)


def _moe_step(sched_ref, x_ref, w_ref, gup_ref, dp_ref, out_ref):
    i = pl.program_id(0)

    @pl.when(i == 0)
    def _init():
        out_ref[...] = jnp.zeros_like(out_ref)

    x = x_ref[...]
    w1e = gup_ref[0]
    gu = jax.lax.dot_general(
        x, w1e, (((1,), (1,)), ((), ())),
        preferred_element_type=jnp.float32)
    gate = gu[:, :DFF]
    up = gu[:, DFF:]
    h = gate * jax.nn.sigmoid(gate) * up
    oe = jax.lax.dot_general(
        h, dp_ref[0], (((1,), (1,)), ((), ())),
        preferred_element_type=jnp.float32)
    e_id = sched_ref[0, i]
    scale = sched_ref[1, i].astype(jnp.float32)
    eidx = jax.lax.broadcasted_iota(jnp.int32, (T, E), 1)
    we = jnp.sum(jnp.where(eidx == e_id, w_ref[...], 0.0), axis=1, keepdims=True)
    out_ref[...] += (scale * we) * oe


@jax.jit
def kernel(x, gating_output, gate_up_proj, down_proj):
    w_te, sched = pl.pallas_call(
        _routing_step,
        in_specs=[pl.BlockSpec((T, E), lambda: (0, 0))],
        out_specs=[
            pl.BlockSpec((T, E), lambda: (0, 0)),
            pl.BlockSpec((8, E), lambda: (0, 0)),
        ],
        out_shape=[
            jax.ShapeDtypeStruct((T, E), jnp.float32),
            jax.ShapeDtypeStruct((8, E), jnp.int32),
        ],
    )(gating_output)

    return pl.pallas_call(
        _moe_step,
        grid_spec=pltpu.PrefetchScalarGridSpec(
            num_scalar_prefetch=1,
            grid=(E,),
            in_specs=[
                pl.BlockSpec((T, D), lambda i, s: (0, 0)),
                pl.BlockSpec((T, E), lambda i, s: (0, 0)),
                pl.BlockSpec((1, 2 * DFF, D), lambda i, s: (s[0, i], 0, 0)),
                pl.BlockSpec((1, D, DFF), lambda i, s: (s[0, i], 0, 0)),
            ],
            out_specs=pl.BlockSpec((T, D), lambda i, s: (0, 0)),
        ),
        out_shape=jax.ShapeDtypeStruct((T, D), jnp.float32),
    )(sched, x, w_te, gate_up_proj, down_proj)
